# baseline (speedup 1.0000x reference)
import functools

import jax
import jax.numpy as jnp
from jax import lax
from jax.experimental import pallas as pl
from jax.experimental.pallas import tpu as pltpu
from jax.experimental.pallas import tpu_sc as plsc

_N_DATA = 1_000_000
_RHO = 0.99
_ALPHA = 0.5
_EPS = 1e-6
_B = 4096
_NC = 2
_NS = 16
_NW = _NC * _NS
_RANGE = _N_DATA // _NW
_NVEC = _B // 16
_NSEG = 4
_SEGV = _NVEC // _NSEG
_SEGPAD = _SEGV * 16 + 16


def _dense_body(feats_ref, c_ref, r_ref, att_ref):
    fa = feats_ref[0:_B, :]
    fb = feats_ref[_B:2 * _B, :]
    fa_roll = jnp.concatenate([fa[1:], fa[:1]], axis=0)
    fb_roll = jnp.concatenate([fb[1:], fb[:1]], axis=0)

    def d2(x):
        return jnp.sum((x * x).reshape(32, 128, 128), axis=2)

    da2 = d2(fa - fb + _EPS)
    db2 = d2(fb - fa + _EPS)
    dra2 = d2(fa - fb_roll + _EPS)
    drb2 = d2(fb - fa_roll + _EPS)
    qa = 1.0 / (1.0 + da2)
    qb = 1.0 / (1.0 + db2)
    qra = 1.0 / (1.0 + dra2)
    qrb = 1.0 / (1.0 + drb2)
    att = (jnp.sum(-jnp.log(qa)) + jnp.sum(-jnp.log(qb))) / (2.0 * _B)
    att_ref[...] = jnp.broadcast_to(att, (1, 1))
    npow2 = jnp.float32(_N_DATA) ** 2
    ema = (1.0 - _RHO) * npow2
    xi_a = _ALPHA * qa + (1.0 - _ALPHA) * qra
    xi_b = _ALPHA * qb + (1.0 - _ALPHA) * qrb
    c_ref[...] = (ema * xi_a + ema * xi_b) * 0.5
    r_ref[...] = qra + qrb


_dense_call = pl.pallas_call(
    _dense_body,
    out_shape=(
        jax.ShapeDtypeStruct((32, 128), jnp.float32),
        jax.ShapeDtypeStruct((32, 128), jnp.float32),
        jax.ShapeDtypeStruct((1, 1), jnp.float32),
    ),
)


_sc_mesh = plsc.VectorSubcoreMesh(
    core_axis_name="c", subcore_axis_name="s", num_cores=_NC, num_subcores=_NS
)


@functools.partial(
    pl.kernel,
    out_type=jax.ShapeDtypeStruct((_NW * 16,), jnp.float32),
    mesh=_sc_mesh,
    compiler_params=pltpu.CompilerParams(needs_layout_passes=False),
    scratch_types=[
        pltpu.VMEM((_B,), jnp.int32),
        pltpu.VMEM((_B,), jnp.float32),
        pltpu.VMEM((_B,), jnp.float32),
        pltpu.VMEM((_NSEG * _SEGPAD,), jnp.int32),
        pltpu.VMEM((_NSEG * _SEGPAD,), jnp.int32),
        pltpu.VMEM((_NSEG * _SEGPAD,), jnp.float32),
        pltpu.VMEM((_NSEG * _SEGPAD,), jnp.float32),
        pltpu.VMEM((_RANGE,), jnp.int32),
        pltpu.VMEM((16,), jnp.float32),
        pltpu.SemaphoreType.DMA,
        pltpu.SemaphoreType.DMA,
        pltpu.SemaphoreType.DMA,
    ],
)
def _sc_update(s_inv_hbm, idx_hbm, c_hbm, r_hbm, sref_hbm, parts_hbm,
               idx_v, c_v, r_v, cidx_v, cpos_v, scur_v, val_v, stamp_v,
               part_v, isem, gsem, ssem):
    wid = lax.axis_index("s") * _NC + lax.axis_index("c")
    obase = wid * _RANGE
    iota = lax.iota(jnp.int32, 16)

    d1 = pltpu.async_copy(idx_hbm, idx_v, isem)
    d2 = pltpu.async_copy(c_hbm, c_v, isem)
    d3 = pltpu.async_copy(r_hbm, r_v, isem)
    d1.wait()
    d2.wait()
    d3.wait()

    def scan_body(i, cnts):
        new = []
        for k in range(_NSEG):
            sl = pl.ds((k * _SEGV + i) * 16, 16)
            iv = idx_v[sl]
            off = iv - obase
            inr = (off >= 0) & (off < _RANGE)
            base = k * _SEGPAD + cnts[k]
            plsc.store_compressed(cidx_v.at[pl.ds(base, 16)], iv, mask=inr)
            plsc.store_compressed(
                cpos_v.at[pl.ds(base, 16)], iota + (k * _SEGV + i) * 16,
                mask=inr)
            new.append(cnts[k] + jnp.sum(inr.astype(jnp.int32)))
        return tuple(new)

    cnts = lax.fori_loop(0, _SEGV, scan_body, (jnp.int32(0),) * _NSEG)
    for k in range(_NSEG):
        cidx_v[pl.ds(k * _SEGPAD + cnts[k], 16)] = jnp.broadcast_to(obase, (16,))
        cpos_v[pl.ds(k * _SEGPAD + cnts[k], 16)] = jnp.zeros((16,), jnp.int32)
    nchs = [(cnts[k] + 15) // 16 for k in range(_NSEG)]

    for k in range(_NSEG):
        def fire_g(j, _, k=k):
            sl = pl.ds(k * _SEGPAD + j * 16, 16)
            pltpu.async_copy(s_inv_hbm.at[cidx_v[sl]], scur_v.at[sl], gsem)
            return 0

        lax.fori_loop(0, nchs[k], fire_g, 0)

    for k in range(_NSEG):
        def drain_g(j, _, k=k):
            pltpu.make_async_copy(
                s_inv_hbm.at[pl.ds(0, 16)],
                scur_v.at[pl.ds(k * _SEGPAD + j * 16, 16)], gsem,
            ).wait()
            return 0

        lax.fori_loop(0, nchs[k], drain_g, 0)

    acc = jnp.zeros((16,), jnp.float32)
    for k in range(_NSEG):
        def comp_body(j, a, k=k):
            sl = pl.ds(k * _SEGPAD + j * 16, 16)
            s_cur = scur_v[sl]
            cval = plsc.load_gather(c_v, [cpos_v[sl]])
            rval = plsc.load_gather(r_v, [cpos_v[sl]])
            valid = (iota + j * 16) < cnts[k]
            val_v[sl] = _RHO * s_cur + cval
            return a + jnp.where(valid, rval / s_cur, 0.0)

        acc = lax.fori_loop(0, nchs[k], comp_body, acc)

    for k in range(_NSEG):
        def stamp_body(j, _, k=k):
            sl = pl.ds(k * _SEGPAD + j * 16, 16)
            ivc = cidx_v[sl]
            valid = (iota + j * 16) < cnts[k]
            _, last = plsc.scan_count(ivc, mask=valid)
            m1 = valid & last
            plsc.store_scatter(stamp_v, [ivc - obase], cpos_v[sl], mask=m1)
            return 0

        lax.fori_loop(0, nchs[k], stamp_body, 0)

    nf = jnp.int32(0)
    for k in range(_NSEG):
        def scat_body(j, f, k=k):
            sl = pl.ds(k * _SEGPAD + j * 16, 16)
            ivc = cidx_v[sl]
            posv = cpos_v[sl]
            valid = (iota + j * 16) < cnts[k]
            _, last = plsc.scan_count(ivc, mask=valid)
            m1 = valid & last
            stamped = plsc.load_gather(stamp_v, [ivc - obase], mask=m1)
            win = m1 & (stamped == posv)
            anyw = jnp.any(win)

            @pl.when(anyw)
            def _():
                vv = val_v[sl]
                wlane = jnp.max(jnp.where(win, iota, -1))
                sel = iota == wlane
                bidx = jnp.sum(jnp.where(sel, ivc, 0))
                bval = jnp.sum(jnp.where(sel, vv, 0.0))
                sidx = jnp.where(win, ivc, bidx)
                val_v[sl] = jnp.where(win, vv, bval)
                pltpu.async_copy(val_v.at[sl], sref_hbm.at[sidx], ssem)

            return f + anyw.astype(jnp.int32)

        nf = lax.fori_loop(0, nchs[k], scat_body, nf)

    def drain_s(j, _):
        pltpu.make_async_copy(
            s_inv_hbm.at[pl.ds(0, 16)], val_v.at[pl.ds(0, 16)], ssem
        ).wait()
        return 0

    lax.fori_loop(0, nf, drain_s, 0)

    part_v[...] = jnp.broadcast_to(jnp.sum(acc), (16,))
    pltpu.sync_copy(part_v, parts_hbm.at[pl.ds(wid * 16, 16)])


def kernel(feats, feats_idx, s_inv):
    c2, r2, att = _dense_call(feats)
    sref = jax.new_ref(s_inv)
    parts = _sc_update(s_inv, feats_idx, c2.reshape(_B), r2.reshape(_B), sref)
    new_s_inv = sref[...]
    npow2 = jnp.float32(_N_DATA) ** 2
    rep = jnp.sum(parts.reshape(_NW, 16)[:, 0]) * (npow2 / jnp.float32(2 * _B))
    loss = att[0, 0] + rep
    return loss, new_s_inv


# speedup vs baseline: 1.1040x; 1.1040x over previous
import functools

import jax
import jax.numpy as jnp
from jax import lax
from jax.experimental import pallas as pl
from jax.experimental.pallas import tpu as pltpu
from jax.experimental.pallas import tpu_sc as plsc

_N_DATA = 1_000_000
_RHO = 0.99
_ALPHA = 0.5
_EPS = 1e-6
_B = 4096
_NC = 2
_NS = 16
_NW = _NC * _NS
_RANGE = _N_DATA // _NW
_NVEC = _B // 16
_PAD = _B + 16


def _dense_body(feats_ref, c_ref, r_ref, att_ref):
    fa = feats_ref[0:_B, :]
    fb = feats_ref[_B:2 * _B, :]
    fa_roll = jnp.concatenate([fa[1:], fa[:1]], axis=0)
    fb_roll = jnp.concatenate([fb[1:], fb[:1]], axis=0)

    def d2(x):
        return jnp.sum((x * x).reshape(32, 128, 128), axis=2)

    da2 = d2(fa - fb + _EPS)
    db2 = d2(fb - fa + _EPS)
    dra2 = d2(fa - fb_roll + _EPS)
    drb2 = d2(fb - fa_roll + _EPS)
    qa = 1.0 / (1.0 + da2)
    qb = 1.0 / (1.0 + db2)
    qra = 1.0 / (1.0 + dra2)
    qrb = 1.0 / (1.0 + drb2)
    att = (jnp.sum(-jnp.log(qa)) + jnp.sum(-jnp.log(qb))) / (2.0 * _B)
    att_ref[...] = jnp.broadcast_to(att, (1, 1))
    npow2 = jnp.float32(_N_DATA) ** 2
    ema = (1.0 - _RHO) * npow2
    xi_a = _ALPHA * qa + (1.0 - _ALPHA) * qra
    xi_b = _ALPHA * qb + (1.0 - _ALPHA) * qrb
    c_ref[...] = (ema * xi_a + ema * xi_b) * 0.5
    r_ref[...] = qra + qrb


_dense_call = pl.pallas_call(
    _dense_body,
    out_shape=(
        jax.ShapeDtypeStruct((32, 128), jnp.float32),
        jax.ShapeDtypeStruct((32, 128), jnp.float32),
        jax.ShapeDtypeStruct((1, 1), jnp.float32),
    ),
)


_sc_mesh = plsc.VectorSubcoreMesh(
    core_axis_name="c", subcore_axis_name="s", num_cores=_NC, num_subcores=_NS
)


@functools.partial(
    pl.kernel,
    out_type=jax.ShapeDtypeStruct((_NW * 16,), jnp.float32),
    mesh=_sc_mesh,
    compiler_params=pltpu.CompilerParams(needs_layout_passes=False),
    scratch_types=[
        pltpu.VMEM((_B,), jnp.int32),
        pltpu.VMEM((_B,), jnp.float32),
        pltpu.VMEM((_B,), jnp.float32),
        pltpu.VMEM((_PAD,), jnp.int32),
        pltpu.VMEM((_PAD,), jnp.int32),
        pltpu.VMEM((_PAD,), jnp.float32),
        pltpu.VMEM((_PAD,), jnp.float32),
        pltpu.VMEM((_RANGE,), jnp.int32),
        pltpu.VMEM((16,), jnp.float32),
        pltpu.SemaphoreType.DMA,
        pltpu.SemaphoreType.DMA,
        pltpu.SemaphoreType.DMA,
    ],
)
def _sc_update(s_inv_hbm, idx_hbm, c_hbm, r_hbm, sref_hbm, parts_hbm,
               idx_v, c_v, r_v, cidx_v, cpos_v, scur_v, val_v, stamp_v,
               part_v, isem, gsem, ssem):
    wid = lax.axis_index("s") * _NC + lax.axis_index("c")
    obase = wid * _RANGE
    iota = lax.iota(jnp.int32, 16)

    with jax.named_scope("stage_in"):
        d1 = pltpu.async_copy(idx_hbm, idx_v, isem)
        d2 = pltpu.async_copy(c_hbm, c_v, isem)
        d3 = pltpu.async_copy(r_hbm, r_v, isem)
        d1.wait()
        d2.wait()
        d3.wait()

    with jax.named_scope("scan"):
        def scan_body(i, cnt):
            sl = pl.ds(i * 16, 16)
            off = idx_v[sl] - obase
            inr = (off >= 0) & (off < _RANGE)
            plsc.store_compressed(cidx_v.at[pl.ds(cnt, 16)], idx_v[sl], mask=inr)
            plsc.store_compressed(cpos_v.at[pl.ds(cnt, 16)], iota + i * 16,
                                  mask=inr)
            return cnt + jnp.sum(inr.astype(jnp.int32))

        cnt = lax.fori_loop(0, _NVEC, scan_body, jnp.int32(0))
        cidx_v[pl.ds(cnt, 16)] = jnp.broadcast_to(obase, (16,))
        cpos_v[pl.ds(cnt, 16)] = jnp.zeros((16,), jnp.int32)
        nch = (cnt + 15) // 16

    with jax.named_scope("gather"):
        def fire_g(j, _):
            sl = pl.ds(j * 16, 16)
            pltpu.async_copy(s_inv_hbm.at[cidx_v[sl]], scur_v.at[sl], gsem)
            return 0

        lax.fori_loop(0, nch, fire_g, 0)

        def drain_g(j, _):
            pltpu.make_async_copy(
                s_inv_hbm.at[pl.ds(0, 16)], scur_v.at[pl.ds(j * 16, 16)], gsem
            ).wait()
            return 0

        lax.fori_loop(0, nch, drain_g, 0)

    with jax.named_scope("compute"):
        def comp_body(j, acc):
            sl = pl.ds(j * 16, 16)
            s_cur = scur_v[sl]
            cval = plsc.load_gather(c_v, [cpos_v[sl]])
            rval = plsc.load_gather(r_v, [cpos_v[sl]])
            valid = (iota + j * 16) < cnt
            val_v[sl] = _RHO * s_cur + cval
            return acc + jnp.where(valid, rval / s_cur, 0.0)

        acc = lax.fori_loop(0, nch, comp_body, jnp.zeros((16,), jnp.float32))

    with jax.named_scope("stamp"):
        def stamp_body(j, _):
            sl = pl.ds(j * 16, 16)
            ivc = cidx_v[sl]
            valid = (iota + j * 16) < cnt
            _, last = plsc.scan_count(ivc, mask=valid)
            m1 = valid & last
            plsc.store_scatter(stamp_v, [ivc - obase], cpos_v[sl], mask=m1)
            return 0

        lax.fori_loop(0, nch, stamp_body, 0)

    with jax.named_scope("scatter"):
        def scat_body(j, nf):
            sl = pl.ds(j * 16, 16)
            ivc = cidx_v[sl]
            posv = cpos_v[sl]
            valid = (iota + j * 16) < cnt
            _, last = plsc.scan_count(ivc, mask=valid)
            m1 = valid & last
            stamped = plsc.load_gather(stamp_v, [ivc - obase], mask=m1)
            win = m1 & (stamped == posv)
            anyw = jnp.any(win)

            @pl.when(anyw)
            def _():
                vv = val_v[sl]
                wlane = jnp.max(jnp.where(win, iota, -1))
                sel = iota == wlane
                bidx = jnp.sum(jnp.where(sel, ivc, 0))
                bval = jnp.sum(jnp.where(sel, vv, 0.0))
                sidx = jnp.where(win, ivc, bidx)
                val_v[sl] = jnp.where(win, vv, bval)
                pltpu.async_copy(val_v.at[sl], sref_hbm.at[sidx], ssem)

            return nf + anyw.astype(jnp.int32)

        nf = lax.fori_loop(0, nch, scat_body, jnp.int32(0))

        def drain_s(j, _):
            pltpu.make_async_copy(
                s_inv_hbm.at[pl.ds(0, 16)], val_v.at[pl.ds(0, 16)], ssem
            ).wait()
            return 0

        lax.fori_loop(0, nf, drain_s, 0)

    with jax.named_scope("parts_out"):
        part_v[...] = jnp.broadcast_to(jnp.sum(acc), (16,))
        pltpu.sync_copy(part_v, parts_hbm.at[pl.ds(wid * 16, 16)])


def kernel(feats, feats_idx, s_inv):
    c2, r2, att = _dense_call(feats)
    sref = jax.new_ref(s_inv)
    parts = _sc_update(s_inv, feats_idx, c2.reshape(_B), r2.reshape(_B), sref)
    new_s_inv = sref[...]
    npow2 = jnp.float32(_N_DATA) ** 2
    rep = jnp.sum(parts.reshape(_NW, 16)[:, 0]) * (npow2 / jnp.float32(2 * _B))
    loss = att[0, 0] + rep
    return loss, new_s_inv


# speedup vs baseline: 1.1597x; 1.0505x over previous
import functools

import jax
import jax.numpy as jnp
from jax import lax
from jax.experimental import pallas as pl
from jax.experimental.pallas import tpu as pltpu
from jax.experimental.pallas import tpu_sc as plsc

_N_DATA = 1_000_000
_RHO = 0.99
_ALPHA = 0.5
_EPS = 1e-6
_B = 4096
_D = 128
_NC = 2
_NS = 16
_NW = _NC * _NS
_RANGE = _N_DATA // _NW
_NVEC = _B // 16
_PAD = _B + 16


def _dense_body(feats_ref, c_ref, r_ref, att_ref):
    fa = feats_ref[0:_B, :]
    fb = feats_ref[_B:2 * _B, :]
    fa_roll = jnp.concatenate([fa[1:], fa[:1]], axis=0)
    fb_roll = jnp.concatenate([fb[1:], fb[:1]], axis=0)
    ones = jnp.ones((_D, 1), jnp.float32)

    def rowsums(x):
        nums = (((0,), (1,)), ((), ()))
        s2 = lax.dot_general(ones, x * x, nums,
                             preferred_element_type=jnp.float32)
        s1 = lax.dot_general(ones, x, nums,
                             preferred_element_type=jnp.float32)
        return s2, s1

    u2, u1 = rowsums(fa - fb)
    v2, v1 = rowsums(fa - fb_roll)
    w2, w1 = rowsums(fb - fa_roll)
    e2 = _D * _EPS * _EPS
    da2 = u2 + 2.0 * _EPS * u1 + e2
    db2 = u2 - 2.0 * _EPS * u1 + e2
    dra2 = v2 + 2.0 * _EPS * v1 + e2
    drb2 = w2 + 2.0 * _EPS * w1 + e2
    qa = 1.0 / (1.0 + da2)
    qb = 1.0 / (1.0 + db2)
    qra = 1.0 / (1.0 + dra2)
    qrb = 1.0 / (1.0 + drb2)
    att = (jnp.sum(-jnp.log(qa)) + jnp.sum(-jnp.log(qb))) / (2.0 * _B)
    att_ref[...] = jnp.broadcast_to(att, (1, 1))
    npow2 = jnp.float32(_N_DATA) ** 2
    ema = (1.0 - _RHO) * npow2
    xi_a = _ALPHA * qa + (1.0 - _ALPHA) * qra
    xi_b = _ALPHA * qb + (1.0 - _ALPHA) * qrb
    c_ref[...] = ((ema * xi_a + ema * xi_b) * 0.5).reshape(_B)
    r_ref[...] = (qra + qrb).reshape(_B)


_dense_call = pl.pallas_call(
    _dense_body,
    out_shape=(
        jax.ShapeDtypeStruct((_B,), jnp.float32),
        jax.ShapeDtypeStruct((_B,), jnp.float32),
        jax.ShapeDtypeStruct((1, 1), jnp.float32),
    ),
)


_sc_mesh = plsc.VectorSubcoreMesh(
    core_axis_name="c", subcore_axis_name="s", num_cores=_NC, num_subcores=_NS
)


@functools.partial(
    pl.kernel,
    out_type=jax.ShapeDtypeStruct((_NW * 16,), jnp.float32),
    mesh=_sc_mesh,
    compiler_params=pltpu.CompilerParams(needs_layout_passes=False),
    scratch_types=[
        pltpu.VMEM((_B,), jnp.int32),
        pltpu.VMEM((_PAD,), jnp.int32),
        pltpu.VMEM((_PAD,), jnp.int32),
        pltpu.VMEM((_PAD,), jnp.float32),
        pltpu.VMEM((_PAD,), jnp.float32),
        pltpu.VMEM((_PAD,), jnp.float32),
        pltpu.VMEM((_PAD,), jnp.float32),
        pltpu.VMEM((_RANGE,), jnp.int32),
        pltpu.VMEM((16,), jnp.float32),
        pltpu.SemaphoreType.DMA,
        pltpu.SemaphoreType.DMA,
        pltpu.SemaphoreType.DMA,
    ],
)
def _sc_update(s_inv_hbm, idx_hbm, c_hbm, r_hbm, sref_hbm, parts_hbm,
               idx_v, cidx_v, cpos_v, scur_v, cval_v, rval_v, val_v, stamp_v,
               part_v, isem, gsem, ssem):
    wid = lax.axis_index("s") * _NC + lax.axis_index("c")
    obase = wid * _RANGE
    iota = lax.iota(jnp.int32, 16)

    with jax.named_scope("stage_in"):
        pltpu.async_copy(idx_hbm, idx_v, isem).wait()

    with jax.named_scope("scan"):
        def scan_body(i, cnt):
            sl = pl.ds(i * 16, 16)
            off = idx_v[sl] - obase
            inr = (off >= 0) & (off < _RANGE)
            plsc.store_compressed(cidx_v.at[pl.ds(cnt, 16)], idx_v[sl], mask=inr)
            plsc.store_compressed(cpos_v.at[pl.ds(cnt, 16)], iota + i * 16,
                                  mask=inr)
            return cnt + jnp.sum(inr.astype(jnp.int32))

        cnt = lax.fori_loop(0, _NVEC, scan_body, jnp.int32(0))
        cidx_v[pl.ds(cnt, 16)] = jnp.broadcast_to(obase, (16,))
        cpos_v[pl.ds(cnt, 16)] = jnp.zeros((16,), jnp.int32)
        nch = (cnt + 15) // 16

    with jax.named_scope("gather"):
        def fire_g(j, _):
            sl = pl.ds(j * 16, 16)
            pltpu.async_copy(s_inv_hbm.at[cidx_v[sl]], scur_v.at[sl], gsem)
            pltpu.async_copy(c_hbm.at[cpos_v[sl]], cval_v.at[sl], gsem)
            pltpu.async_copy(r_hbm.at[cpos_v[sl]], rval_v.at[sl], gsem)
            return 0

        lax.fori_loop(0, nch, fire_g, 0)

        def drain_g(j, _):
            pltpu.make_async_copy(
                s_inv_hbm.at[pl.ds(0, 16)], scur_v.at[pl.ds(j * 16, 16)], gsem
            ).wait()
            return 0

        lax.fori_loop(0, 3 * nch, drain_g, 0)

    with jax.named_scope("compute"):
        def comp_body(j, acc):
            sl = pl.ds(j * 16, 16)
            s_cur = scur_v[sl]
            valid = (iota + j * 16) < cnt
            val_v[sl] = _RHO * s_cur + cval_v[sl]
            return acc + jnp.where(valid, rval_v[sl] / s_cur, 0.0)

        acc = lax.fori_loop(0, nch, comp_body, jnp.zeros((16,), jnp.float32))

    with jax.named_scope("stamp"):
        def stamp_body(j, _):
            sl = pl.ds(j * 16, 16)
            ivc = cidx_v[sl]
            valid = (iota + j * 16) < cnt
            _, last = plsc.scan_count(ivc, mask=valid)
            m1 = valid & last
            plsc.store_scatter(stamp_v, [ivc - obase], cpos_v[sl], mask=m1)
            return 0

        lax.fori_loop(0, nch, stamp_body, 0)

    with jax.named_scope("scatter"):
        def scat_body(j, nf):
            sl = pl.ds(j * 16, 16)
            ivc = cidx_v[sl]
            posv = cpos_v[sl]
            valid = (iota + j * 16) < cnt
            _, last = plsc.scan_count(ivc, mask=valid)
            m1 = valid & last
            stamped = plsc.load_gather(stamp_v, [ivc - obase], mask=m1)
            win = m1 & (stamped == posv)
            anyw = jnp.any(win)

            @pl.when(anyw)
            def _():
                vv = val_v[sl]
                wlane = jnp.max(jnp.where(win, iota, -1))
                sel = iota == wlane
                bidx = jnp.sum(jnp.where(sel, ivc, 0))
                bval = jnp.sum(jnp.where(sel, vv, 0.0))
                sidx = jnp.where(win, ivc, bidx)
                val_v[sl] = jnp.where(win, vv, bval)
                pltpu.async_copy(val_v.at[sl], sref_hbm.at[sidx], ssem)

            return nf + anyw.astype(jnp.int32)

        nf = lax.fori_loop(0, nch, scat_body, jnp.int32(0))

        def drain_s(j, _):
            pltpu.make_async_copy(
                s_inv_hbm.at[pl.ds(0, 16)], val_v.at[pl.ds(0, 16)], ssem
            ).wait()
            return 0

        lax.fori_loop(0, nf, drain_s, 0)

    with jax.named_scope("parts_out"):
        part_v[...] = jnp.broadcast_to(jnp.sum(acc), (16,))
        pltpu.sync_copy(part_v, parts_hbm.at[pl.ds(wid * 16, 16)])


def kernel(feats, feats_idx, s_inv):
    c, r, att = _dense_call(feats)
    sref = jax.new_ref(s_inv)
    parts = _sc_update(s_inv, feats_idx, c, r, sref)
    new_s_inv = sref[...]
    npow2 = jnp.float32(_N_DATA) ** 2
    rep = jnp.sum(parts.reshape(_NW, 16)[:, 0]) * (npow2 / jnp.float32(2 * _B))
    loss = att[0, 0] + rep
    return loss, new_s_inv


# speedup vs baseline: 1.3910x; 1.1994x over previous
import functools

import jax
import jax.numpy as jnp
from jax import lax
from jax.experimental import pallas as pl
from jax.experimental.pallas import tpu as pltpu
from jax.experimental.pallas import tpu_sc as plsc

_N_DATA = 1_000_000
_RHO = 0.99
_ALPHA = 0.5
_EPS = 1e-6
_B = 4096
_D = 128
_NC = 2
_NS = 16
_NW = _NC * _NS
_SLICE = 31248
_TAIL = _N_DATA - _NW * _SLICE
_RMAX = _SLICE + _TAIL
_NVEC = _B // 16
_PAD = _B + 16


def _dense_body(feats_ref, c_ref, r_ref, att_ref):
    fa = feats_ref[0:_B, :]
    fb = feats_ref[_B:2 * _B, :]
    fa_roll = jnp.concatenate([fa[1:], fa[:1]], axis=0)
    fb_roll = jnp.concatenate([fb[1:], fb[:1]], axis=0)
    ones = jnp.ones((_D, 1), jnp.float32)

    def rowsums(x):
        nums = (((0,), (1,)), ((), ()))
        s2 = lax.dot_general(ones, x * x, nums,
                             preferred_element_type=jnp.float32)
        s1 = lax.dot_general(ones, x, nums,
                             preferred_element_type=jnp.float32)
        return s2, s1

    u2, u1 = rowsums(fa - fb)
    v2, v1 = rowsums(fa - fb_roll)
    w2, w1 = rowsums(fb - fa_roll)
    e2 = _D * _EPS * _EPS
    da2 = u2 + 2.0 * _EPS * u1 + e2
    db2 = u2 - 2.0 * _EPS * u1 + e2
    dra2 = v2 + 2.0 * _EPS * v1 + e2
    drb2 = w2 + 2.0 * _EPS * w1 + e2
    qa = 1.0 / (1.0 + da2)
    qb = 1.0 / (1.0 + db2)
    qra = 1.0 / (1.0 + dra2)
    qrb = 1.0 / (1.0 + drb2)
    att = (jnp.sum(-jnp.log(qa)) + jnp.sum(-jnp.log(qb))) / (2.0 * _B)
    att_ref[...] = jnp.broadcast_to(att, (1, 1))
    npow2 = jnp.float32(_N_DATA) ** 2
    ema = (1.0 - _RHO) * npow2
    xi_a = _ALPHA * qa + (1.0 - _ALPHA) * qra
    xi_b = _ALPHA * qb + (1.0 - _ALPHA) * qrb
    c_ref[...] = ((ema * xi_a + ema * xi_b) * 0.5).reshape(_B)
    r_ref[...] = (qra + qrb).reshape(_B)


_dense_call = pl.pallas_call(
    _dense_body,
    out_shape=(
        jax.ShapeDtypeStruct((_B,), jnp.float32),
        jax.ShapeDtypeStruct((_B,), jnp.float32),
        jax.ShapeDtypeStruct((1, 1), jnp.float32),
    ),
)


_sc_mesh = plsc.VectorSubcoreMesh(
    core_axis_name="c", subcore_axis_name="s", num_cores=_NC, num_subcores=_NS
)


@functools.partial(
    pl.kernel,
    out_type=(
        jax.ShapeDtypeStruct((_N_DATA,), jnp.float32),
        jax.ShapeDtypeStruct((_NW * 16,), jnp.float32),
    ),
    mesh=_sc_mesh,
    compiler_params=pltpu.CompilerParams(needs_layout_passes=False),
    scratch_types=[
        pltpu.VMEM((_RMAX,), jnp.float32),
        pltpu.VMEM((_RMAX,), jnp.int32),
        pltpu.VMEM((_B,), jnp.int32),
        pltpu.VMEM((_PAD,), jnp.int32),
        pltpu.VMEM((_PAD,), jnp.int32),
        pltpu.VMEM((_PAD,), jnp.float32),
        pltpu.VMEM((_PAD,), jnp.float32),
        pltpu.VMEM((_PAD,), jnp.float32),
        pltpu.VMEM((16,), jnp.float32),
        pltpu.SemaphoreType.DMA,
        pltpu.SemaphoreType.DMA,
        pltpu.SemaphoreType.DMA,
    ],
)
def _sc_update(s_inv_hbm, idx_hbm, c_hbm, r_hbm, out_hbm, parts_hbm,
               slice_v, stamp_v, idx_v, cidx_v, cpos_v, cval_v, rval_v, val_v,
               part_v, isem, csem, gsem):
    wid = lax.axis_index("s") * _NC + lax.axis_index("c")
    obase = wid * _SLICE
    osize = jnp.where(wid == _NW - 1, _RMAX, _SLICE)
    iota = lax.iota(jnp.int32, 16)

    with jax.named_scope("fire_copy"):
        d_slice = pltpu.async_copy(
            s_inv_hbm.at[pl.ds(obase, _SLICE)],
            slice_v.at[pl.ds(0, _SLICE)], csem)

        @pl.when(wid == _NW - 1)
        def _():
            pltpu.async_copy(
                s_inv_hbm.at[pl.ds(_NW * _SLICE, _TAIL)],
                slice_v.at[pl.ds(_SLICE, _TAIL)], csem)

    with jax.named_scope("stage_in"):
        pltpu.async_copy(idx_hbm, idx_v, isem).wait()

    with jax.named_scope("scan"):
        def scan_body(i, cnt):
            sl = pl.ds(i * 16, 16)
            off = idx_v[sl] - obase
            inr = (off >= 0) & (off < osize)
            plsc.store_compressed(cidx_v.at[pl.ds(cnt, 16)], idx_v[sl], mask=inr)
            plsc.store_compressed(cpos_v.at[pl.ds(cnt, 16)], iota + i * 16,
                                  mask=inr)
            return cnt + jnp.sum(inr.astype(jnp.int32))

        cnt = lax.fori_loop(0, _NVEC, scan_body, jnp.int32(0))
        cidx_v[pl.ds(cnt, 16)] = jnp.broadcast_to(obase, (16,))
        cpos_v[pl.ds(cnt, 16)] = jnp.zeros((16,), jnp.int32)
        nch = (cnt + 15) // 16

    with jax.named_scope("gather"):
        def fire_g(j, _):
            sl = pl.ds(j * 16, 16)
            pltpu.async_copy(c_hbm.at[cpos_v[sl]], cval_v.at[sl], gsem)
            pltpu.async_copy(r_hbm.at[cpos_v[sl]], rval_v.at[sl], gsem)
            return 0

        lax.fori_loop(0, nch, fire_g, 0)

        def drain_g(j, _):
            pltpu.make_async_copy(
                s_inv_hbm.at[pl.ds(0, 16)], cval_v.at[pl.ds(j * 16, 16)], gsem
            ).wait()
            return 0

        lax.fori_loop(0, 2 * nch, drain_g, 0)

    with jax.named_scope("drain_copy"):
        d_slice.wait()

        @pl.when(wid == _NW - 1)
        def _():
            pltpu.make_async_copy(
                s_inv_hbm.at[pl.ds(_NW * _SLICE, _TAIL)],
                slice_v.at[pl.ds(_SLICE, _TAIL)], csem,
            ).wait()

    with jax.named_scope("compute"):
        def comp_body(j, acc):
            sl = pl.ds(j * 16, 16)
            off = cidx_v[sl] - obase
            valid = (iota + j * 16) < cnt
            s_cur = plsc.load_gather(slice_v, [off], mask=valid)
            val_v[sl] = _RHO * s_cur + cval_v[sl]
            return acc + jnp.where(valid, rval_v[sl] / s_cur, 0.0)

        acc = lax.fori_loop(0, nch, comp_body, jnp.zeros((16,), jnp.float32))

    with jax.named_scope("stamp"):
        def stamp_body(j, _):
            sl = pl.ds(j * 16, 16)
            ivc = cidx_v[sl]
            valid = (iota + j * 16) < cnt
            _, last = plsc.scan_count(ivc, mask=valid)
            m1 = valid & last
            plsc.store_scatter(stamp_v, [ivc - obase], cpos_v[sl], mask=m1)
            return 0

        lax.fori_loop(0, nch, stamp_body, 0)

    with jax.named_scope("scatter"):
        def scat_body(j, _):
            sl = pl.ds(j * 16, 16)
            ivc = cidx_v[sl]
            posv = cpos_v[sl]
            valid = (iota + j * 16) < cnt
            _, last = plsc.scan_count(ivc, mask=valid)
            m1 = valid & last
            stamped = plsc.load_gather(stamp_v, [ivc - obase], mask=m1)
            win = m1 & (stamped == posv)
            plsc.store_scatter(slice_v, [ivc - obase], val_v[sl], mask=win)
            return 0

        lax.fori_loop(0, nch, scat_body, 0)

    with jax.named_scope("copy_out"):
        d_out = pltpu.async_copy(
            slice_v.at[pl.ds(0, _SLICE)],
            out_hbm.at[pl.ds(obase, _SLICE)], csem)

        @pl.when(wid == _NW - 1)
        def _():
            pltpu.async_copy(
                slice_v.at[pl.ds(_SLICE, _TAIL)],
                out_hbm.at[pl.ds(_NW * _SLICE, _TAIL)], csem)

    with jax.named_scope("parts_out"):
        part_v[...] = jnp.broadcast_to(jnp.sum(acc), (16,))
        pltpu.sync_copy(part_v, parts_hbm.at[pl.ds(wid * 16, 16)])

    with jax.named_scope("drain_out"):
        d_out.wait()

        @pl.when(wid == _NW - 1)
        def _():
            pltpu.make_async_copy(
                slice_v.at[pl.ds(_SLICE, _TAIL)],
                out_hbm.at[pl.ds(_NW * _SLICE, _TAIL)], csem,
            ).wait()


def kernel(feats, feats_idx, s_inv):
    c, r, att = _dense_call(feats)
    new_s_inv, parts = _sc_update(s_inv, feats_idx, c, r)
    npow2 = jnp.float32(_N_DATA) ** 2
    rep = jnp.sum(parts.reshape(_NW, 16)[:, 0]) * (npow2 / jnp.float32(2 * _B))
    loss = att[0, 0] + rep
    return loss, new_s_inv
